# Initial kernel scaffold; baseline (speedup 1.0000x reference)
import jax
import jax.numpy as jnp
from jax.experimental import pallas as pl


def kernel(u_pos, v_pos, v_neg, visual_pos, batch_size, u_emb, v_emb, visual_table, gate_W, gate_b):
    raise NotImplementedError("write your pallas kernel here")


# speedup vs baseline: 4.6480x; 4.6480x over previous
import functools

import jax
import jax.numpy as jnp
import numpy as np
from jax import lax
from jax.experimental import pallas as pl
from jax.experimental.pallas import tpu as pltpu
from jax.experimental.pallas import tpu_sc as plsc

_NC = 2
_NS = 16
_NW = _NC * _NS
_L = 16

_EMB = 64
_NEG = 20
_DC = _EMB // _L

_GDN = lax.GatherDimensionNumbers(
    offset_dims=(), collapsed_slice_dims=(0,), start_index_map=(0,))


def _hsum_all_lanes(v, lanes):
    for s in (1, 2, 4, 8):
        perm = jnp.reshape(lanes ^ s, (_L, 1))
        p = lax.gather(v, perm, _GDN, slice_sizes=(1,),
                       mode=lax.GatherScatterMode.PROMISE_IN_BOUNDS)
        v = v + p
    return v


def _sc_scores(u_pos, v_pos, v_neg, u_emb, v_emb):
    B = u_pos.shape[0]
    bpw = B // _NW
    n_gj = bpw // 128
    neg_gj = (bpw * _NEG) // 128
    CB = 32
    n_chunks = bpw // CB
    gpc = (CB * _NEG) // 128

    u_pos3 = u_pos.reshape(_NW, n_gj, 128)
    v_pos3 = v_pos.reshape(_NW, n_chunks, CB)
    v_neg3 = v_neg.reshape(_NW, neg_gj, 128)

    mesh = plsc.VectorSubcoreMesh(core_axis_name="c", subcore_axis_name="s")

    @functools.partial(
        pl.kernel,
        mesh=mesh,
        compiler_params=pltpu.CompilerParams(use_tc_tiling_on_sc=False),
        out_type=[
            jax.ShapeDtypeStruct((B,), jnp.float32),
            jax.ShapeDtypeStruct((B,), jnp.float32),
        ],
        scratch_types=[
            pltpu.VMEM((n_gj, 128), jnp.int32),
            pltpu.VMEM((n_chunks, CB), jnp.int32),
            pltpu.VMEM((neg_gj, 128), jnp.int32),
            pltpu.VMEM((bpw, _EMB), jnp.float32),
            pltpu.VMEM((CB, _EMB), jnp.float32),
            pltpu.VMEM((CB * _NEG, _EMB), jnp.float32),
            pltpu.VMEM((bpw,), jnp.float32),
            pltpu.VMEM((bpw,), jnp.float32),
        ],
    )
    def sc_kernel(u_pos_hbm, v_pos_hbm, v_neg_hbm, u_emb_hbm, v_emb_hbm,
                  score_hbm, negscore_hbm,
                  uidx_v, vidx_v, nidx_v, u_rows, v_buf, neg_buf,
                  score_v, negs_v):
        wid = lax.axis_index("s") * _NC + lax.axis_index("c")
        base = wid * bpw

        pltpu.sync_copy(u_pos_hbm.at[wid], uidx_v)
        pltpu.sync_copy(v_pos_hbm.at[wid], vidx_v)
        pltpu.sync_copy(v_neg_hbm.at[wid], nidx_v)

        for j in range(n_gj):
            pltpu.sync_copy(u_emb_hbm.at[uidx_v.at[j]],
                            u_rows.at[pl.ds(j * 128, 128)])

        lanes = lax.iota(jnp.int32, _L)
        zeros = jnp.zeros((_L,), jnp.float32)

        def chunk_body(c, _):
            pltpu.sync_copy(v_emb_hbm.at[vidx_v.at[c]], v_buf)
            for j in range(gpc):
                pltpu.sync_copy(v_emb_hbm.at[nidx_v.at[c * gpc + j]],
                                neg_buf.at[pl.ds(j * 128, 128)])

            def pos_body(g, _):
                vec = zeros
                for j in range(_L):
                    bb = g * _L + j
                    b = c * CB + bb
                    t = zeros
                    for dc in range(_DC):
                        t = t + (u_rows[b, pl.ds(dc * _L, _L)]
                                 * v_buf[bb, pl.ds(dc * _L, _L)])
                    vec = jnp.where(lanes == j, _hsum_all_lanes(t, lanes),
                                    vec)
                score_v[pl.ds((c * CB + g * _L), _L)] = vec
                return 0

            lax.fori_loop(0, CB // _L, pos_body, 0)

            def neg_body(g, _):
                vec = zeros
                for j in range(_L):
                    bb = g * _L + j
                    r = bb * _NEG
                    b = c * CB + bb
                    t = zeros
                    for dc in range(_DC):
                        a = neg_buf[r, pl.ds(dc * _L, _L)]
                        for k in range(1, _NEG):
                            a = a + neg_buf[r + k, pl.ds(dc * _L, _L)]
                        t = t + a * u_rows[b, pl.ds(dc * _L, _L)]
                    vec = jnp.where(lanes == j, _hsum_all_lanes(t, lanes),
                                    vec)
                negs_v[pl.ds((c * CB + g * _L), _L)] = vec
                return 0

            lax.fori_loop(0, CB // _L, neg_body, 0)
            return 0

        lax.fori_loop(0, n_chunks, chunk_body, 0)

        pltpu.sync_copy(score_v, score_hbm.at[pl.ds(base, bpw)])
        pltpu.sync_copy(negs_v, negscore_hbm.at[pl.ds(base, bpw)])

    return sc_kernel(u_pos3, v_pos3, v_neg3, u_emb, v_emb)


def _tc_loss(score, neg_score):
    B = score.shape[0]
    s2 = score.reshape(128, B // 128)
    n2 = neg_score.reshape(128, B // 128)

    def body(s_ref, n_ref, o_ref):
        s = s_ref[...]
        n = n_ref[...]
        ls = jnp.minimum(s, 0.0) - jnp.log1p(jnp.exp(-jnp.abs(s)))
        ln = jnp.minimum(-n, 0.0) - jnp.log1p(jnp.exp(-jnp.abs(n)))
        o_ref[...] = jnp.reshape((jnp.sum(ls) + jnp.sum(ln)) * (-1.0 / B),
                                 (1, 1))

    out = pl.pallas_call(
        body,
        out_shape=jax.ShapeDtypeStruct((1, 1), jnp.float32),
    )(s2, n2)
    return out.reshape(())


def kernel(u_pos, v_pos, v_neg, visual_pos, batch_size,
           u_emb, v_emb, visual_table, gate_W, gate_b):
    score, neg_score = _sc_scores(u_pos, v_pos, v_neg, u_emb, v_emb)
    return _tc_loss(score, neg_score)


# speedup vs baseline: 7.2514x; 1.5601x over previous
import functools

import jax
import jax.numpy as jnp
from jax import lax
from jax.experimental import pallas as pl
from jax.experimental.pallas import tpu as pltpu
from jax.experimental.pallas import tpu_sc as plsc

_NC = 2
_NS = 16
_NW = _NC * _NS
_L = 16

_EMB = 64
_NEG = 20
_DC = _EMB // _L

_GDN = lax.GatherDimensionNumbers(
    offset_dims=(), collapsed_slice_dims=(0,), start_index_map=(0,))


def _hsum_all_lanes(v, lanes):
    for s in (1, 2, 4, 8):
        perm = jnp.reshape(lanes ^ s, (_L, 1))
        p = lax.gather(v, perm, _GDN, slice_sizes=(1,),
                       mode=lax.GatherScatterMode.PROMISE_IN_BOUNDS)
        v = v + p
    return v


def _prep_table(u_emb, v_emb):
    V = u_emb.shape[0]
    R = 7936
    grid = pl.cdiv(V, R)

    def body(u_ref, v_ref, o_ref):
        o_ref[:, 0:_EMB] = u_ref[...].T
        o_ref[:, _EMB:2 * _EMB] = v_ref[...].T

    return pl.pallas_call(
        body,
        grid=(grid,),
        in_specs=[
            pl.BlockSpec((_EMB, R), lambda i: (0, i)),
            pl.BlockSpec((_EMB, R), lambda i: (0, i)),
        ],
        out_specs=pl.BlockSpec((R, 2 * _EMB), lambda i: (i, 0)),
        out_shape=jax.ShapeDtypeStruct((V, 2 * _EMB), jnp.float32),
    )(u_emb.T, v_emb.T)


def _sc_scores(u_pos, v_pos, v_neg, w_table):
    B = u_pos.shape[0]
    bpw = B // _NW
    CB = 16
    n_chunks = bpw // CB
    gpc = (CB * _NEG) // 64
    neg_gj = n_chunks * gpc

    u_pos3 = u_pos.reshape(_NW, n_chunks, CB)
    v_pos3 = v_pos.reshape(_NW, n_chunks, CB)
    v_neg3 = v_neg.reshape(_NW, neg_gj, 64)

    mesh = plsc.VectorSubcoreMesh(core_axis_name="c", subcore_axis_name="s")

    @functools.partial(
        pl.kernel,
        mesh=mesh,
        compiler_params=pltpu.CompilerParams(use_tc_tiling_on_sc=True),
        out_type=[
            jax.ShapeDtypeStruct((B,), jnp.float32),
            jax.ShapeDtypeStruct((B,), jnp.float32),
        ],
        scratch_types=[
            pltpu.VMEM((n_chunks, CB), jnp.int32),
            pltpu.VMEM((n_chunks, CB), jnp.int32),
            pltpu.VMEM((neg_gj, 64), jnp.int32),
            pltpu.VMEM((CB, 2 * _EMB), jnp.float32),
            pltpu.VMEM((CB, 2 * _EMB), jnp.float32),
            pltpu.VMEM((CB * _NEG, 2 * _EMB), jnp.float32),
            pltpu.VMEM((bpw,), jnp.float32),
            pltpu.VMEM((bpw,), jnp.float32),
        ],
    )
    def sc_kernel(u_pos_hbm, v_pos_hbm, v_neg_hbm, w_hbm,
                  score_hbm, negscore_hbm,
                  uidx_v, vidx_v, nidx_v, u_buf, v_buf, neg_buf,
                  score_v, negs_v):
        wid = lax.axis_index("s") * _NC + lax.axis_index("c")
        base = wid * bpw

        pltpu.sync_copy(u_pos_hbm.at[wid], uidx_v)
        pltpu.sync_copy(v_pos_hbm.at[wid], vidx_v)
        pltpu.sync_copy(v_neg_hbm.at[wid], nidx_v)

        lanes = lax.iota(jnp.int32, _L)
        zeros = jnp.zeros((_L,), jnp.float32)

        def chunk_body(c, _):
            pltpu.sync_copy(w_hbm.at[uidx_v.at[c]], u_buf)
            pltpu.sync_copy(w_hbm.at[vidx_v.at[c]], v_buf)
            for j in range(gpc):
                pltpu.sync_copy(w_hbm.at[nidx_v.at[c * gpc + j]],
                                neg_buf.at[pl.ds(j * 64, 64)])

            svec = zeros
            nvec = zeros
            for j in range(_L):
                r = j * _NEG
                t = zeros
                tn = zeros
                for dc in range(_DC):
                    ud = u_buf[j, pl.ds(dc * _L, _L)]
                    t = t + ud * v_buf[j, pl.ds(_EMB + dc * _L, _L)]
                    a = neg_buf[r, pl.ds(_EMB + dc * _L, _L)]
                    for k in range(1, _NEG):
                        a = a + neg_buf[r + k, pl.ds(_EMB + dc * _L, _L)]
                    tn = tn + a * ud
                svec = jnp.where(lanes == j, _hsum_all_lanes(t, lanes), svec)
                nvec = jnp.where(lanes == j, _hsum_all_lanes(tn, lanes),
                                 nvec)
            score_v[pl.ds(c * CB, CB)] = svec
            negs_v[pl.ds(c * CB, CB)] = nvec
            return 0

        lax.fori_loop(0, n_chunks, chunk_body, 0)

        pltpu.sync_copy(score_v, score_hbm.at[pl.ds(base, bpw)])
        pltpu.sync_copy(negs_v, negscore_hbm.at[pl.ds(base, bpw)])

    return sc_kernel(u_pos3, v_pos3, v_neg3, w_table)


def _tc_loss(score, neg_score):
    B = score.shape[0]
    s2 = score.reshape(128, B // 128)
    n2 = neg_score.reshape(128, B // 128)

    def body(s_ref, n_ref, o_ref):
        s = s_ref[...]
        n = n_ref[...]
        ls = jnp.minimum(s, 0.0) - jnp.log1p(jnp.exp(-jnp.abs(s)))
        ln = jnp.minimum(-n, 0.0) - jnp.log1p(jnp.exp(-jnp.abs(n)))
        o_ref[...] = jnp.reshape((jnp.sum(ls) + jnp.sum(ln)) * (-1.0 / B),
                                 (1, 1))

    out = pl.pallas_call(
        body,
        out_shape=jax.ShapeDtypeStruct((1, 1), jnp.float32),
    )(s2, n2)
    return out.reshape(())


def kernel(u_pos, v_pos, v_neg, visual_pos, batch_size,
           u_emb, v_emb, visual_table, gate_W, gate_b):
    w_table = _prep_table(u_emb, v_emb)
    score, neg_score = _sc_scores(u_pos, v_pos, v_neg, w_table)
    return _tc_loss(score, neg_score)


# speedup vs baseline: 11.2335x; 1.5491x over previous
import functools

import jax
import jax.numpy as jnp
from jax import lax
from jax.experimental import pallas as pl
from jax.experimental.pallas import tpu as pltpu
from jax.experimental.pallas import tpu_sc as plsc

_NC = 2
_NS = 16
_NW = _NC * _NS
_L = 16

_EMB = 64
_NEG = 20
_DC = _EMB // _L

_GDN = lax.GatherDimensionNumbers(
    offset_dims=(), collapsed_slice_dims=(0,), start_index_map=(0,))


def _hsum_all_lanes(v, lanes):
    for s in (1, 2, 4, 8):
        perm = jnp.reshape(lanes ^ s, (_L, 1))
        p = lax.gather(v, perm, _GDN, slice_sizes=(1,),
                       mode=lax.GatherScatterMode.PROMISE_IN_BOUNDS)
        v = v + p
    return v


def _prep_table(u_emb, v_emb):
    V = u_emb.shape[0]
    R = 7936
    grid = pl.cdiv(V, R)

    def body(u_ref, v_ref, o_ref):
        o_ref[:, 0:_EMB] = u_ref[...].T
        o_ref[:, _EMB:2 * _EMB] = v_ref[...].T

    return pl.pallas_call(
        body,
        grid=(grid,),
        in_specs=[
            pl.BlockSpec((_EMB, R), lambda i: (0, i)),
            pl.BlockSpec((_EMB, R), lambda i: (0, i)),
        ],
        out_specs=pl.BlockSpec((R, 2 * _EMB), lambda i: (i, 0)),
        out_shape=jax.ShapeDtypeStruct((V, 2 * _EMB), jnp.float32),
    )(u_emb.T, v_emb.T)


def _sc_scores(u_pos, v_pos, v_neg, w_table):
    B = u_pos.shape[0]
    bpw = B // _NW
    CB = 16
    n_chunks = bpw // CB
    gpc = (CB * _NEG) // 64
    neg_gj = n_chunks * gpc

    u_pos3 = u_pos.reshape(_NW, n_chunks, CB)
    v_pos3 = v_pos.reshape(_NW, n_chunks, CB)
    v_neg3 = v_neg.reshape(_NW, neg_gj, 64)

    mesh = plsc.VectorSubcoreMesh(core_axis_name="c", subcore_axis_name="s")

    @functools.partial(
        pl.kernel,
        mesh=mesh,
        compiler_params=pltpu.CompilerParams(use_tc_tiling_on_sc=True),
        out_type=[
            jax.ShapeDtypeStruct((B,), jnp.float32),
            jax.ShapeDtypeStruct((B,), jnp.float32),
        ],
        scratch_types=[
            pltpu.VMEM((n_chunks, CB), jnp.int32),
            pltpu.VMEM((n_chunks, CB), jnp.int32),
            pltpu.VMEM((neg_gj, 64), jnp.int32),
            pltpu.VMEM((2, CB, 2 * _EMB), jnp.float32),
            pltpu.VMEM((2, CB, 2 * _EMB), jnp.float32),
            pltpu.VMEM((2, CB * _NEG, 2 * _EMB), jnp.float32),
            pltpu.VMEM((bpw,), jnp.float32),
            pltpu.VMEM((bpw,), jnp.float32),
            pltpu.SemaphoreType.DMA,
            pltpu.SemaphoreType.DMA,
        ],
    )
    def sc_kernel(u_pos_hbm, v_pos_hbm, v_neg_hbm, w_hbm,
                  score_hbm, negscore_hbm,
                  uidx_v, vidx_v, nidx_v, u_buf2, v_buf2, neg_buf2,
                  score_v, negs_v, sem_a, sem_b):
        wid = lax.axis_index("s") * _NC + lax.axis_index("c")
        base = wid * bpw

        pltpu.sync_copy(u_pos_hbm.at[wid], uidx_v)
        pltpu.sync_copy(v_pos_hbm.at[wid], vidx_v)
        pltpu.sync_copy(v_neg_hbm.at[wid], nidx_v)

        lanes = lax.iota(jnp.int32, _L)
        zeros = jnp.zeros((_L,), jnp.float32)

        def copies(c, p, sem):
            yield w_hbm.at[uidx_v.at[c]], u_buf2.at[p], sem
            yield w_hbm.at[vidx_v.at[c]], v_buf2.at[p], sem
            for j in range(gpc):
                yield (w_hbm.at[nidx_v.at[c * gpc + j]],
                       neg_buf2.at[p, pl.ds(j * 64, 64)], sem)

        def fire(c, p, sem):
            for src, dst, s in copies(c, p, sem):
                pltpu.async_copy(src, dst, s)

        def drain(c, p, sem):
            for src, dst, s in copies(c, p, sem):
                pltpu.make_async_copy(src, dst, s).wait()

        def compute(c, p):
            u_buf = u_buf2.at[p]
            v_buf = v_buf2.at[p]
            neg_buf = neg_buf2.at[p]

            def j_body(j, carry):
                svec, nvec = carry
                r = j * _NEG
                t = zeros
                tn = zeros
                for dc in range(_DC):
                    ud = u_buf[j, pl.ds(dc * _L, _L)]
                    t = t + ud * v_buf[j, pl.ds(_EMB + dc * _L, _L)]
                    a = neg_buf[r, pl.ds(_EMB + dc * _L, _L)]
                    for k in range(1, _NEG):
                        a = a + neg_buf[r + k, pl.ds(_EMB + dc * _L, _L)]
                    tn = tn + a * ud
                svec = jnp.where(lanes == j, _hsum_all_lanes(t, lanes), svec)
                nvec = jnp.where(lanes == j, _hsum_all_lanes(tn, lanes),
                                 nvec)
                return svec, nvec

            svec, nvec = lax.fori_loop(0, _L, j_body, (zeros, zeros))
            score_v[pl.ds(c * CB, CB)] = svec
            negs_v[pl.ds(c * CB, CB)] = nvec

        fire(0, 0, sem_a)

        def pair_body(c2, _):
            c0 = 2 * c2
            c1 = c0 + 1
            fire(c1, 1, sem_b)
            drain(c0, 0, sem_a)
            compute(c0, 0)

            @pl.when(c2 + 1 < n_chunks // 2)
            def _():
                fire(c0 + 2, 0, sem_a)

            drain(c1, 1, sem_b)
            compute(c1, 1)
            return 0

        lax.fori_loop(0, n_chunks // 2, pair_body, 0)

        pltpu.sync_copy(score_v, score_hbm.at[pl.ds(base, bpw)])
        pltpu.sync_copy(negs_v, negscore_hbm.at[pl.ds(base, bpw)])

    return sc_kernel(u_pos3, v_pos3, v_neg3, w_table)


def _tc_loss(score, neg_score):
    B = score.shape[0]
    s2 = score.reshape(128, B // 128)
    n2 = neg_score.reshape(128, B // 128)

    def body(s_ref, n_ref, o_ref):
        s = s_ref[...]
        n = n_ref[...]
        ls = jnp.minimum(s, 0.0) - jnp.log1p(jnp.exp(-jnp.abs(s)))
        ln = jnp.minimum(-n, 0.0) - jnp.log1p(jnp.exp(-jnp.abs(n)))
        o_ref[...] = jnp.reshape((jnp.sum(ls) + jnp.sum(ln)) * (-1.0 / B),
                                 (1, 1))

    out = pl.pallas_call(
        body,
        out_shape=jax.ShapeDtypeStruct((1, 1), jnp.float32),
    )(s2, n2)
    return out.reshape(())


def kernel(u_pos, v_pos, v_neg, visual_pos, batch_size,
           u_emb, v_emb, visual_table, gate_W, gate_b):
    w_table = _prep_table(u_emb, v_emb)
    score, neg_score = _sc_scores(u_pos, v_pos, v_neg, w_table)
    return _tc_loss(score, neg_score)


# speedup vs baseline: 14.0702x; 1.2525x over previous
import functools

import jax
import jax.numpy as jnp
from jax import lax
from jax.experimental import pallas as pl
from jax.experimental.pallas import tpu as pltpu
from jax.experimental.pallas import tpu_sc as plsc

_NC = 2
_NS = 16
_NW = _NC * _NS
_L = 16

_EMB = 64
_NEG = 20
_DC = _EMB // _L

_GDN = lax.GatherDimensionNumbers(
    offset_dims=(), collapsed_slice_dims=(0,), start_index_map=(0,))


def _hsum_all_lanes(v, lanes):
    for s in (1, 2, 4, 8):
        perm = jnp.reshape(lanes ^ s, (_L, 1))
        p = lax.gather(v, perm, _GDN, slice_sizes=(1,),
                       mode=lax.GatherScatterMode.PROMISE_IN_BOUNDS)
        v = v + p
    return v


def _prep_table(u_emb, v_emb):
    V = u_emb.shape[0]
    R = 7936
    grid = pl.cdiv(V, R)

    def body(u_ref, v_ref, o_ref):
        o_ref[...] = jnp.concatenate([u_ref[...], v_ref[...]], axis=0).T

    return pl.pallas_call(
        body,
        grid=(grid,),
        in_specs=[
            pl.BlockSpec((_EMB, R), lambda i: (0, i)),
            pl.BlockSpec((_EMB, R), lambda i: (0, i)),
        ],
        out_specs=pl.BlockSpec((R, 2 * _EMB), lambda i: (i, 0)),
        out_shape=jax.ShapeDtypeStruct((V, 2 * _EMB), jnp.float32),
    )(u_emb.T, v_emb.T)


def _sc_scores(u_pos, v_pos, v_neg, w_table):
    B = u_pos.shape[0]
    bpw = B // _NW
    CB = 16
    n_chunks = bpw // CB
    gpc = (CB * _NEG) // 64
    neg_gj = n_chunks * gpc

    u_pos3 = u_pos.reshape(_NW, n_chunks, CB)
    v_pos3 = v_pos.reshape(_NW, n_chunks, CB)
    v_neg3 = v_neg.reshape(_NW, neg_gj, 64)

    mesh = plsc.VectorSubcoreMesh(core_axis_name="c", subcore_axis_name="s")

    @functools.partial(
        pl.kernel,
        mesh=mesh,
        compiler_params=pltpu.CompilerParams(use_tc_tiling_on_sc=True),
        out_type=[
            jax.ShapeDtypeStruct((B,), jnp.float32),
            jax.ShapeDtypeStruct((B,), jnp.float32),
        ],
        scratch_types=[
            pltpu.VMEM((n_chunks, CB), jnp.int32),
            pltpu.VMEM((n_chunks, CB), jnp.int32),
            pltpu.VMEM((neg_gj, 64), jnp.int32),
            pltpu.VMEM((2, CB, 2 * _EMB), jnp.float32),
            pltpu.VMEM((2, CB, 2 * _EMB), jnp.float32),
            pltpu.VMEM((2, CB * _NEG, 2 * _EMB), jnp.float32),
            pltpu.VMEM((bpw,), jnp.float32),
            pltpu.VMEM((bpw,), jnp.float32),
            pltpu.SemaphoreType.DMA,
            pltpu.SemaphoreType.DMA,
        ],
    )
    def sc_kernel(u_pos_hbm, v_pos_hbm, v_neg_hbm, w_hbm,
                  score_hbm, negscore_hbm,
                  uidx_v, vidx_v, nidx_v, u_buf2, v_buf2, neg_buf2,
                  score_v, negs_v, sem_a, sem_b):
        wid = lax.axis_index("s") * _NC + lax.axis_index("c")
        base = wid * bpw

        pltpu.sync_copy(u_pos_hbm.at[wid], uidx_v)
        pltpu.sync_copy(v_pos_hbm.at[wid], vidx_v)
        pltpu.sync_copy(v_neg_hbm.at[wid], nidx_v)

        lanes = lax.iota(jnp.int32, _L)
        zeros = jnp.zeros((_L,), jnp.float32)

        def copies(c, p, sem):
            yield w_hbm.at[uidx_v.at[c]], u_buf2.at[p], sem
            yield w_hbm.at[vidx_v.at[c]], v_buf2.at[p], sem
            for j in range(gpc):
                yield (w_hbm.at[nidx_v.at[c * gpc + j]],
                       neg_buf2.at[p, pl.ds(j * 64, 64)], sem)

        def fire(c, p, sem):
            for src, dst, s in copies(c, p, sem):
                pltpu.async_copy(src, dst, s)

        def drain(c, p, sem):
            for src, dst, s in copies(c, p, sem):
                pltpu.make_async_copy(src, dst, s).wait()

        def compute(c, p):
            u_buf = u_buf2.at[p]
            v_buf = v_buf2.at[p]
            neg_buf = neg_buf2.at[p]

            def j_body(j, carry):
                svec, nvec = carry
                r = j * _NEG
                t = zeros
                tn = zeros
                for dc in range(_DC):
                    ud = u_buf[j, pl.ds(dc * _L, _L)]
                    t = t + ud * v_buf[j, pl.ds(_EMB + dc * _L, _L)]
                    a = neg_buf[r, pl.ds(_EMB + dc * _L, _L)]
                    for k in range(1, _NEG):
                        a = a + neg_buf[r + k, pl.ds(_EMB + dc * _L, _L)]
                    tn = tn + a * ud
                svec = jnp.where(lanes == j, _hsum_all_lanes(t, lanes), svec)
                nvec = jnp.where(lanes == j, _hsum_all_lanes(tn, lanes),
                                 nvec)
                return svec, nvec

            svec, nvec = lax.fori_loop(0, _L, j_body, (zeros, zeros))
            score_v[pl.ds(c * CB, CB)] = svec
            negs_v[pl.ds(c * CB, CB)] = nvec

        fire(0, 0, sem_a)

        def pair_body(c2, _):
            c0 = 2 * c2
            c1 = c0 + 1
            fire(c1, 1, sem_b)
            drain(c0, 0, sem_a)
            compute(c0, 0)

            @pl.when(c2 + 1 < n_chunks // 2)
            def _():
                fire(c0 + 2, 0, sem_a)

            drain(c1, 1, sem_b)
            compute(c1, 1)
            return 0

        lax.fori_loop(0, n_chunks // 2, pair_body, 0)

        pltpu.sync_copy(score_v, score_hbm.at[pl.ds(base, bpw)])
        pltpu.sync_copy(negs_v, negscore_hbm.at[pl.ds(base, bpw)])

    return sc_kernel(u_pos3, v_pos3, v_neg3, w_table)


def _tc_loss(score, neg_score):
    B = score.shape[0]
    s2 = score.reshape(128, B // 128)
    n2 = neg_score.reshape(128, B // 128)

    def body(s_ref, n_ref, o_ref):
        s = s_ref[...]
        n = n_ref[...]
        ls = jnp.minimum(s, 0.0) - jnp.log1p(jnp.exp(-jnp.abs(s)))
        ln = jnp.minimum(-n, 0.0) - jnp.log1p(jnp.exp(-jnp.abs(n)))
        o_ref[...] = jnp.reshape((jnp.sum(ls) + jnp.sum(ln)) * (-1.0 / B),
                                 (1, 1))

    out = pl.pallas_call(
        body,
        out_shape=jax.ShapeDtypeStruct((1, 1), jnp.float32),
    )(s2, n2)
    return out.reshape(())


def kernel(u_pos, v_pos, v_neg, visual_pos, batch_size,
           u_emb, v_emb, visual_table, gate_W, gate_b):
    w_table = _prep_table(u_emb, v_emb)
    score, neg_score = _sc_scores(u_pos, v_pos, v_neg, w_table)
    return _tc_loss(score, neg_score)


# speedup vs baseline: 15.4119x; 1.0954x over previous
import functools

import jax
import jax.numpy as jnp
from jax import lax
from jax.experimental import pallas as pl
from jax.experimental.pallas import tpu as pltpu
from jax.experimental.pallas import tpu_sc as plsc

_NC = 2
_NS = 16
_NW = _NC * _NS
_L = 16

_EMB = 64
_NEG = 20
_DC = _EMB // _L

_GDN = lax.GatherDimensionNumbers(
    offset_dims=(), collapsed_slice_dims=(0,), start_index_map=(0,))


def _hsum_all_lanes(v, lanes):
    for s in (1, 2, 4, 8):
        perm = jnp.reshape(lanes ^ s, (_L, 1))
        p = lax.gather(v, perm, _GDN, slice_sizes=(1,),
                       mode=lax.GatherScatterMode.PROMISE_IN_BOUNDS)
        v = v + p
    return v


def _prep_table(u_emb, v_emb):
    V = u_emb.shape[0]
    R = 7936
    grid = pl.cdiv(V, R)

    def body(u_ref, v_ref, o_ref):
        o_ref[...] = jnp.concatenate([u_ref[...], v_ref[...]], axis=0).T

    return pl.pallas_call(
        body,
        grid=(grid,),
        in_specs=[
            pl.BlockSpec((_EMB, R), lambda i: (0, i)),
            pl.BlockSpec((_EMB, R), lambda i: (0, i)),
        ],
        out_specs=pl.BlockSpec((R, 2 * _EMB), lambda i: (i, 0)),
        out_shape=jax.ShapeDtypeStruct((V, 2 * _EMB), jnp.float32),
    )(u_emb.T, v_emb.T)


def _sc_scores(u_pos, v_pos, v_neg, w2_table):
    B = u_pos.shape[0]
    bpw = B // _NW
    CB = 16
    n_chunks = bpw // CB
    gpc = (CB * _NEG) // 64
    neg_gj = n_chunks * gpc

    u_pos3 = u_pos.reshape(_NW, n_chunks, CB)
    v_pos3 = v_pos.reshape(_NW, n_chunks, CB)
    v_neg3 = v_neg.reshape(_NW, neg_gj, 64)

    mesh = plsc.VectorSubcoreMesh(core_axis_name="c", subcore_axis_name="s")

    @functools.partial(
        pl.kernel,
        mesh=mesh,
        compiler_params=pltpu.CompilerParams(use_tc_tiling_on_sc=False),
        out_type=[
            jax.ShapeDtypeStruct((B,), jnp.float32),
            jax.ShapeDtypeStruct((B,), jnp.float32),
        ],
        scratch_types=[
            pltpu.VMEM((n_chunks, CB), jnp.int32),
            pltpu.VMEM((n_chunks, CB), jnp.int32),
            pltpu.VMEM((neg_gj, 64), jnp.int32),
            pltpu.VMEM((2, CB, _EMB), jnp.float32),
            pltpu.VMEM((2, CB, _EMB), jnp.float32),
            pltpu.VMEM((2, CB * _NEG, _EMB), jnp.float32),
            pltpu.VMEM((bpw,), jnp.float32),
            pltpu.VMEM((bpw,), jnp.float32),
            pltpu.SemaphoreType.DMA,
            pltpu.SemaphoreType.DMA,
        ],
    )
    def sc_kernel(u_pos_hbm, v_pos_hbm, v_neg_hbm, w_hbm,
                  score_hbm, negscore_hbm,
                  uidx_v, vidx_v, nidx_v, u_buf2, v_buf2, neg_buf2,
                  score_v, negs_v, sem_a, sem_b):
        wid = lax.axis_index("s") * _NC + lax.axis_index("c")
        base = wid * bpw

        pltpu.sync_copy(u_pos_hbm.at[wid], uidx_v)
        pltpu.sync_copy(v_pos_hbm.at[wid], vidx_v)
        pltpu.sync_copy(v_neg_hbm.at[wid], nidx_v)

        lanes = lax.iota(jnp.int32, _L)
        zeros = jnp.zeros((_L,), jnp.float32)

        def copies(c, p, sem):
            yield w_hbm.at[uidx_v.at[c]], u_buf2.at[p], sem
            yield w_hbm.at[vidx_v.at[c]], v_buf2.at[p], sem
            for j in range(gpc):
                yield (w_hbm.at[nidx_v.at[c * gpc + j]],
                       neg_buf2.at[p, pl.ds(j * 64, 64)], sem)

        def fire(c, p, sem):
            for src, dst, s in copies(c, p, sem):
                pltpu.async_copy(src, dst, s)

        def drain(c, p, sem):
            for src, dst, s in copies(c, p, sem):
                pltpu.make_async_copy(src, dst, s).wait()

        def compute(c, p):
            u_buf = u_buf2.at[p]
            v_buf = v_buf2.at[p]
            neg_buf = neg_buf2.at[p]

            def j_body(j, carry):
                svec, nvec = carry
                r = j * _NEG
                t = zeros
                tn = zeros
                for dc in range(_DC):
                    ud = u_buf[j, pl.ds(dc * _L, _L)]
                    t = t + ud * v_buf[j, pl.ds(dc * _L, _L)]
                    a = neg_buf[r, pl.ds(dc * _L, _L)]
                    for k in range(1, _NEG):
                        a = a + neg_buf[r + k, pl.ds(dc * _L, _L)]
                    tn = tn + a * ud
                svec = jnp.where(lanes == j, _hsum_all_lanes(t, lanes), svec)
                nvec = jnp.where(lanes == j, _hsum_all_lanes(tn, lanes),
                                 nvec)
                return svec, nvec

            svec, nvec = lax.fori_loop(0, _L, j_body, (zeros, zeros))
            score_v[pl.ds(c * CB, CB)] = svec
            negs_v[pl.ds(c * CB, CB)] = nvec

        fire(0, 0, sem_a)

        def pair_body(c2, _):
            c0 = 2 * c2
            c1 = c0 + 1
            fire(c1, 1, sem_b)
            drain(c0, 0, sem_a)
            compute(c0, 0)

            @pl.when(c2 + 1 < n_chunks // 2)
            def _():
                fire(c0 + 2, 0, sem_a)

            drain(c1, 1, sem_b)
            compute(c1, 1)
            return 0

        lax.fori_loop(0, n_chunks // 2, pair_body, 0)

        pltpu.sync_copy(score_v, score_hbm.at[pl.ds(base, bpw)])
        pltpu.sync_copy(negs_v, negscore_hbm.at[pl.ds(base, bpw)])

    return sc_kernel(u_pos3, v_pos3, v_neg3, w2_table)


def _tc_loss(score, neg_score):
    B = score.shape[0]
    s2 = score.reshape(128, B // 128)
    n2 = neg_score.reshape(128, B // 128)

    def body(s_ref, n_ref, o_ref):
        s = s_ref[...]
        n = n_ref[...]
        ls = jnp.minimum(s, 0.0) - jnp.log1p(jnp.exp(-jnp.abs(s)))
        ln = jnp.minimum(-n, 0.0) - jnp.log1p(jnp.exp(-jnp.abs(n)))
        o_ref[...] = jnp.reshape((jnp.sum(ls) + jnp.sum(ln)) * (-1.0 / B),
                                 (1, 1))

    out = pl.pallas_call(
        body,
        out_shape=jax.ShapeDtypeStruct((1, 1), jnp.float32),
    )(s2, n2)
    return out.reshape(())


def kernel(u_pos, v_pos, v_neg, visual_pos, batch_size,
           u_emb, v_emb, visual_table, gate_W, gate_b):
    w_table = _prep_table(u_emb, v_emb)
    w2 = w_table.reshape(2 * w_table.shape[0], _EMB)
    score, neg_score = _sc_scores(2 * u_pos, 2 * v_pos + 1, 2 * v_neg + 1,
                                  w2)
    return _tc_loss(score, neg_score)
